# baseline (speedup 1.0000x reference)
import functools

import jax
import jax.numpy as jnp
from jax import lax
from jax.experimental import pallas as pl
from jax.experimental.pallas import tpu as pltpu
from jax.experimental.pallas import tpu_sc as plsc

N_V = 6890
NS = 16
L = 16
R = 432
G = R // L
TAIL_W = NS - 1
TAIL_VALID = N_V - TAIL_W * R
TAIL_G = (TAIL_VALID + L - 1) // L
TAIL_REM = TAIL_VALID - (TAIL_G - 1) * L
SCALE = 1.0 / (N_V * 3)

_mesh = plsc.VectorSubcoreMesh(
    core_axis_name="c", subcore_axis_name="s", num_cores=1
)


@functools.partial(
    pl.kernel,
    mesh=_mesh,
    compiler_params=pltpu.CompilerParams(
        needs_layout_passes=False, use_tc_tiling_on_sc=False
    ),
    out_type=jax.ShapeDtypeStruct((1,), jnp.float32),
    scratch_types=[
        pltpu.VMEM((N_V,), jnp.float32),
        pltpu.VMEM((N_V,), jnp.float32),
        pltpu.VMEM((N_V,), jnp.float32),
        pltpu.VMEM((R,), jnp.int32),
        pltpu.VMEM((R,), jnp.float32),
        pltpu.VMEM((L,), jnp.float32),
        pltpu.VMEM((NS, L), jnp.float32),
        pltpu.HBM((NS, L), jnp.float32),
        pltpu.SemaphoreType.DMA,
    ],
)
def _sym_loss_kernel(vt_hbm, w_hbm, idx_hbm, out_hbm,
                     xt, yt, zt, idx_v, w_v, stage_v, acc_v, partials, sem):
    s = lax.axis_index("s")
    base = s * R

    lanes = lax.iota(jnp.int32, L)
    zeros_i = jnp.zeros((L,), jnp.int32)
    zeros_f = jnp.zeros((L,), jnp.float32)

    def stream_tables():
        return [
            pltpu.async_copy(vt_hbm.at[0], xt, sem),
            pltpu.async_copy(vt_hbm.at[1], yt, sem),
            pltpu.async_copy(vt_hbm.at[2], zt, sem),
        ]

    def group_contrib(t, rows, rl):
        idx16 = idx_v[pl.ds(t * L, L)]
        vx = plsc.load_gather(xt, [rows])
        vy = plsc.load_gather(yt, [rows])
        vz = plsc.load_gather(zt, [rows])
        mx = plsc.load_gather(xt, [idx16])
        my = plsc.load_gather(yt, [idx16])
        mz = plsc.load_gather(zt, [idx16])
        w16 = w_v[pl.ds(t * L, L)]
        dx = vx + mx
        dy = vy - my
        dz = vz - mz
        return w16 * (dx * dx + dy * dy + dz * dz)

    @pl.when(s < TAIL_W)
    def _full_tile():
        copies = stream_tables()
        pltpu.sync_copy(idx_hbm.at[pl.ds(base, R)], idx_v)
        pltpu.sync_copy(w_hbm.at[pl.ds(base, R)], w_v)
        for cp in copies:
            cp.wait()
        acc = jnp.zeros((L,), jnp.float32)
        for t in range(G):
            rl = lanes + t * L
            acc = acc + group_contrib(t, rl + base, rl)
        stage_v[...] = acc

    @pl.when(s == TAIL_W)
    def _tail_tile():
        copies = stream_tables()
        pre = (TAIL_VALID // L) * L
        idx_v[pl.ds(pre, L)] = zeros_i
        idx_v[pl.ds(pre + L, L)] = zeros_i
        tbase = TAIL_W * R
        pltpu.sync_copy(idx_hbm.at[pl.ds(tbase, TAIL_VALID)],
                        idx_v.at[pl.ds(0, TAIL_VALID)])
        pltpu.sync_copy(w_hbm.at[pl.ds(tbase, TAIL_VALID)],
                        w_v.at[pl.ds(0, TAIL_VALID)])
        for cp in copies:
            cp.wait()
        acc = jnp.zeros((L,), jnp.float32)
        for t in range(TAIL_G - 1):
            rl = lanes + t * L
            acc = acc + group_contrib(t, rl + tbase, rl)
        last = TAIL_G - 1
        rl = jnp.minimum(lanes + last * L, TAIL_VALID - 1)
        contrib = group_contrib(last, rl + tbase, rl)
        acc = acc + jnp.where(lanes < TAIL_REM, contrib, zeros_f)
        stage_v[...] = acc

    pltpu.sync_copy(stage_v, partials.at[s])
    plsc.subcore_barrier()

    @pl.when(s == 0)
    def _leader():
        pltpu.sync_copy(partials, acc_v)
        vec = acc_v[0]
        for i in range(1, NS):
            vec = vec + acc_v[i]
        total = jnp.sum(vec) * SCALE
        stage_v[...] = jnp.full((L,), total, jnp.float32)
        pltpu.sync_copy(stage_v.at[pl.ds(0, 1)], out_hbm)


def kernel(v, symmetry_w, idx):
    out = _sym_loss_kernel(v.T, symmetry_w.reshape(-1),
                           idx.astype(jnp.int32))
    return out.reshape(())


# speedup vs baseline: 1.0013x; 1.0013x over previous
import functools

import jax
import jax.numpy as jnp
from jax import lax
from jax.experimental import pallas as pl
from jax.experimental.pallas import tpu as pltpu
from jax.experimental.pallas import tpu_sc as plsc

N_V = 6890
SEG = 6896
X_OFF, Y_OFF, Z_OFF = 0, SEG, 2 * SEG
W_OFF, I_OFF = 3 * SEG, 4 * SEG
PACKED = 4 * SEG + N_V
NS = 16
L = 16
R = 432
G = R // L
TAIL_W = NS - 1
TAIL_VALID = N_V - TAIL_W * R
TAIL_G = (TAIL_VALID + L - 1) // L
TAIL_REM = TAIL_VALID - (TAIL_G - 1) * L
SCALE = 1.0 / (N_V * 3)

_mesh = plsc.VectorSubcoreMesh(
    core_axis_name="c", subcore_axis_name="s", num_cores=1
)


@functools.partial(
    pl.kernel,
    mesh=_mesh,
    compiler_params=pltpu.CompilerParams(
        needs_layout_passes=False, use_tc_tiling_on_sc=False
    ),
    out_type=jax.ShapeDtypeStruct((1,), jnp.float32),
    scratch_types=[
        pltpu.VMEM((N_V,), jnp.float32),
        pltpu.VMEM((N_V,), jnp.float32),
        pltpu.VMEM((N_V,), jnp.float32),
        pltpu.VMEM((R,), jnp.float32),
        pltpu.VMEM((R,), jnp.float32),
        pltpu.VMEM((L,), jnp.float32),
        pltpu.VMEM((NS, L), jnp.float32),
        pltpu.HBM((NS, L), jnp.float32),
        pltpu.SemaphoreType.DMA,
    ],
)
def _sym_loss_kernel(buf_hbm, out_hbm,
                     xt, yt, zt, idx_v, w_v, stage_v, acc_v, partials, sem):
    s = lax.axis_index("s")
    base = s * R

    lanes = lax.iota(jnp.int32, L)
    zeros_f = jnp.zeros((L,), jnp.float32)

    def stream_tables():
        return [
            pltpu.async_copy(buf_hbm.at[pl.ds(X_OFF, N_V)], xt, sem),
            pltpu.async_copy(buf_hbm.at[pl.ds(Y_OFF, N_V)], yt, sem),
            pltpu.async_copy(buf_hbm.at[pl.ds(Z_OFF, N_V)], zt, sem),
        ]

    def group_contrib(t, rows, rl):
        idx16 = plsc.bitcast(idx_v[pl.ds(t * L, L)], jnp.int32)
        vx = plsc.load_gather(xt, [rows])
        vy = plsc.load_gather(yt, [rows])
        vz = plsc.load_gather(zt, [rows])
        mx = plsc.load_gather(xt, [idx16])
        my = plsc.load_gather(yt, [idx16])
        mz = plsc.load_gather(zt, [idx16])
        w16 = w_v[pl.ds(t * L, L)]
        dx = vx + mx
        dy = vy - my
        dz = vz - mz
        return w16 * (dx * dx + dy * dy + dz * dz)

    @pl.when(s < TAIL_W)
    def _full_tile():
        copies = stream_tables()
        pltpu.sync_copy(buf_hbm.at[pl.ds(I_OFF + base, R)], idx_v)
        pltpu.sync_copy(buf_hbm.at[pl.ds(W_OFF + base, R)], w_v)
        for cp in copies:
            cp.wait()
        acc = jnp.zeros((L,), jnp.float32)
        for t in range(G):
            rl = lanes + t * L
            acc = acc + group_contrib(t, rl + base, rl)
        stage_v[...] = acc

    @pl.when(s == TAIL_W)
    def _tail_tile():
        copies = stream_tables()
        pre = (TAIL_VALID // L) * L
        idx_v[pl.ds(pre, L)] = zeros_f
        idx_v[pl.ds(pre + L, L)] = zeros_f
        tbase = TAIL_W * R
        pltpu.sync_copy(buf_hbm.at[pl.ds(I_OFF + tbase, TAIL_VALID)],
                        idx_v.at[pl.ds(0, TAIL_VALID)])
        pltpu.sync_copy(buf_hbm.at[pl.ds(W_OFF + tbase, TAIL_VALID)],
                        w_v.at[pl.ds(0, TAIL_VALID)])
        for cp in copies:
            cp.wait()
        acc = jnp.zeros((L,), jnp.float32)
        for t in range(TAIL_G - 1):
            rl = lanes + t * L
            acc = acc + group_contrib(t, rl + tbase, rl)
        last = TAIL_G - 1
        rl = jnp.minimum(lanes + last * L, TAIL_VALID - 1)
        contrib = group_contrib(last, rl + tbase, rl)
        acc = acc + jnp.where(lanes < TAIL_REM, contrib, zeros_f)
        stage_v[...] = acc

    pltpu.sync_copy(stage_v, partials.at[s])
    plsc.subcore_barrier()

    @pl.when(s == 0)
    def _leader():
        pltpu.sync_copy(partials, acc_v)
        vec = acc_v[0]
        for i in range(1, NS):
            vec = vec + acc_v[i]
        total = jnp.sum(vec) * SCALE
        stage_v[...] = jnp.full((L,), total, jnp.float32)
        pltpu.sync_copy(stage_v.at[pl.ds(0, 1)], out_hbm)


def kernel(v, symmetry_w, idx):
    vt = v.T
    pad6 = jnp.zeros((SEG - N_V,), jnp.float32)
    packed = jnp.concatenate([
        vt[0], pad6, vt[1], pad6, vt[2], pad6,
        symmetry_w.reshape(-1), pad6,
        jax.lax.bitcast_convert_type(idx.astype(jnp.int32), jnp.float32),
    ])
    out = _sym_loss_kernel(packed)
    return out.reshape(())
